# baseline (speedup 1.0000x reference)
import numpy as np

import jax
import jax.numpy as jnp
from jax import lax
from jax.experimental import pallas as pl
from jax.experimental.pallas import tpu as pltpu

_MIN_KEPT = 131072
_NLOG07_BITS = int(np.float32(-np.log(np.float32(0.7))).view(np.int32))
_INF_BITS = int(np.float32(np.inf).view(np.int32))
_EDGE_BITS = 26
_EDGES = [(j + 1) * (1 << _EDGE_BITS) - 1 for j in range(32)]


def _make_body(num_steps, rank):
    def body(pred_ref, tgt_ref, out_ref, scr, acc):
        i = pl.program_id(0)
        x = pred_ref[0]
        lab = tgt_ref[0]
        bh = x.shape[1]
        m = jnp.max(x, axis=0)
        s = jnp.sum(jnp.exp(x - m[None, :, :]), axis=0)
        cls = lax.broadcasted_iota(jnp.int32, x.shape, 0)
        xm_lab = jnp.sum(jnp.where(cls == lab[None, :, :], x, 0.0), axis=0) - m
        nll = jnp.log(s) - xm_lab
        b = lax.bitcast_convert_type(nll, jnp.int32)
        scr[pl.ds(i * bh, bh)] = b

        @pl.when(i == 0)
        def _():
            acc[...] = jnp.zeros_like(acc)

        for j, e in enumerate(_EDGES):
            acc[j] = acc[j] + jnp.sum(
                jnp.where(b <= jnp.int32(e), 1.0, 0.0), axis=0)

        @pl.when(i == num_steps - 1)
        def _():
            k = jnp.float32(rank)
            v = scr[...]

            cnt32 = jnp.sum(acc[...], axis=1)
            mlt = jnp.sum(jnp.where(cnt32 < k, 1, 0)).astype(jnp.int32)
            lo0 = mlt * jnp.int32(1 << _EDGE_BITS)
            hi0 = lo0 + jnp.int32((1 << _EDGE_BITS) - 1)

            def search(_, carry):
                lo, hi = carry
                mid = lo + lax.div(hi - lo, jnp.int32(2))
                cnt = jnp.sum(jnp.sum(jnp.where(v <= mid, 1.0, 0.0), axis=0))
                ge = cnt >= k
                return jnp.where(ge, lo, mid + 1), jnp.where(ge, mid, hi)

            _, kth = lax.fori_loop(0, _EDGE_BITS, search, (lo0, hi0))
            thr = jnp.minimum(kth, jnp.int32(_NLOG07_BITS))
            mask = v >= thr
            vf = lax.bitcast_convert_type(v, jnp.float32)
            cnt = jnp.sum(jnp.where(mask, 1.0, 0.0))
            tot = jnp.sum(jnp.where(mask, vf, 0.0))
            out_ref[0, 0] = tot / jnp.maximum(cnt, 1.0)

    return body


def kernel(predict, target):
    n, c, h, w = predict.shape
    bh = 512
    hb = h // bh
    num_steps = n * hb
    npix = n * h * w
    rank = npix - _MIN_KEPT + 1

    out = pl.pallas_call(
        _make_body(num_steps, rank),
        grid=(num_steps,),
        in_specs=[
            pl.BlockSpec((1, c, bh, w), lambda i: (i // hb, 0, i % hb, 0)),
            pl.BlockSpec((1, bh, w), lambda i: (i // hb, i % hb, 0)),
        ],
        out_specs=pl.BlockSpec(memory_space=pltpu.SMEM),
        out_shape=jax.ShapeDtypeStruct((1, 1), jnp.float32),
        scratch_shapes=[pltpu.VMEM((npix // w, w), jnp.int32),
                        pltpu.VMEM((32, w), jnp.float32)],
    )(predict, target)
    return out[0, 0]


# speedup vs baseline: 1.1941x; 1.1941x over previous
import numpy as np

import jax
import jax.numpy as jnp
from jax import lax
from jax.experimental import pallas as pl
from jax.experimental.pallas import tpu as pltpu

_MIN_KEPT = 131072
_NLOG07_BITS = int(np.float32(-np.log(np.float32(0.7))).view(np.int32))
_INF_BITS = int(np.float32(np.inf).view(np.int32))


def _make_body(num_steps, rank):
    def body(pred_ref, tgt_ref, out_ref, scr):
        i = pl.program_id(0)
        x = pred_ref[0]
        lab = tgt_ref[0]
        bh = x.shape[1]
        m = jnp.max(x, axis=0)
        s = jnp.sum(jnp.exp(x - m[None, :, :]), axis=0)
        cls = lax.broadcasted_iota(jnp.int32, x.shape, 0)
        xm_lab = jnp.sum(jnp.where(cls == lab[None, :, :], x, 0.0), axis=0) - m
        nll = jnp.log(s) - xm_lab
        scr[pl.ds(i * bh, bh)] = lax.bitcast_convert_type(nll, jnp.int32)

        @pl.when(i == num_steps - 1)
        def _():
            k = jnp.float32(rank)
            v = scr[...]

            def search(_, carry):
                lo, hi = carry
                mid = lo + lax.div(hi - lo, jnp.int32(2))
                cnt = jnp.sum(jnp.sum(jnp.where(v <= mid, 1.0, 0.0), axis=0))
                ge = cnt >= k
                return jnp.where(ge, lo, mid + 1), jnp.where(ge, mid, hi)

            _, kth = lax.fori_loop(
                0, 31, search, (jnp.int32(0), jnp.int32(_INF_BITS)))
            thr = jnp.minimum(kth, jnp.int32(_NLOG07_BITS))
            mask = v >= thr
            vf = lax.bitcast_convert_type(v, jnp.float32)
            cnt = jnp.sum(jnp.where(mask, 1.0, 0.0))
            tot = jnp.sum(jnp.where(mask, vf, 0.0))
            out_ref[0, 0] = tot / jnp.maximum(cnt, 1.0)

    return body


def kernel(predict, target):
    n, c, h, w = predict.shape
    bh = 512
    hb = h // bh
    num_steps = n * hb
    npix = n * h * w
    rank = npix - _MIN_KEPT + 1

    out = pl.pallas_call(
        _make_body(num_steps, rank),
        grid=(num_steps,),
        in_specs=[
            pl.BlockSpec((1, c, bh, w), lambda i: (i // hb, 0, i % hb, 0)),
            pl.BlockSpec((1, bh, w), lambda i: (i // hb, i % hb, 0)),
        ],
        out_specs=pl.BlockSpec(memory_space=pltpu.SMEM),
        out_shape=jax.ShapeDtypeStruct((1, 1), jnp.float32),
        scratch_shapes=[pltpu.VMEM((npix // w, w), jnp.int32)],
    )(predict, target)
    return out[0, 0]
